# baseline (speedup 1.0000x reference)
import functools

import jax
import jax.numpy as jnp
from jax import lax
from jax.experimental import pallas as pl
from jax.experimental.pallas import tpu as pltpu
from jax.experimental.pallas import tpu_sc as plsc

_N = 10000
_D = 256
_HF = 128
_E = 160000
_CH = 128
_NS = 16
_NCH = 80
_EP = _NS * _NCH * _CH
_NH = 5000
_ACC = 5120
_TRASH = _NH

_BR = 1000
_NG = _N // _BR


def _fill(buf, val, rows, cols):
    npc = cols // 16
    v = jnp.full((16,), val, jnp.float32)

    def body(i, carry):
        r = i // npc
        co = pl.multiple_of((i % npc) * 16, 16)
        buf[r, pl.ds(co, 16)] = v
        return carry

    lax.fori_loop(0, rows * npc, body, 0)


@functools.cache
def _sc_scatter_kernel():
    return functools.partial(
        pl.kernel,
        mesh=plsc.VectorSubcoreMesh(core_axis_name="c", subcore_axis_name="s"),
        out_type=[jax.ShapeDtypeStruct((_NH, _HF), jnp.float32),
                  jax.ShapeDtypeStruct((_NH, _HF), jnp.float32)],
        scratch_types=[
            pltpu.VMEM((_NCH, _CH), jnp.int32),
            pltpu.VMEM((_NCH, _CH), jnp.int32),
            pltpu.VMEM((3 * _CH, _HF), jnp.float32),
            pltpu.SemaphoreType.DMA,
            pltpu.VMEM_SHARED((_ACC, _HF), jnp.float32),
            pltpu.SemaphoreType.DMA,
        ],
    )(_sc_scatter_body)


def _sc_scatter(p0, p1, src3, dst3_0, dst3_1):
    return _sc_scatter_kernel()(p0, p1, src3, dst3_0, dst3_1)


def _sc_scatter_body(p0, p1, src3, dst3_0, dst3_1, out0, out1,
                     src_v, dst_v, buf, sem1, acc, sem0):
    c = lax.axis_index("c")
    s = lax.axis_index("s")

    pltpu.sync_copy(src3.at[s], src_v)

    _fill(buf, 0.0, _CH, _HF)
    zb = s * 320
    for off, sz in ((0, 128), (128, 128), (256, 64)):
        pltpu.sync_copy(buf.at[pl.ds(0, sz)], acc.at[pl.ds(zb + off, sz)])
    plsc.subcore_barrier()

    def run(tab, dst3, out):
        pltpu.sync_copy(dst3.at[s], dst_v)
        pltpu.async_copy(tab.at[src_v.at[0]], buf.at[pl.ds(0, _CH)], sem0)

        def body(i, carry):
            @pl.when(i >= 2)
            def _():
                pltpu.make_async_copy(buf.at[pl.ds(0, _CH)],
                                      acc.at[pl.ds(0, _CH)], sem1).wait()
            nxt = jnp.minimum(i + 1, _NCH - 1)
            off_n = pl.multiple_of(((i + 1) % 3) * _CH, _CH)
            pltpu.async_copy(tab.at[src_v.at[nxt]], buf.at[pl.ds(off_n, _CH)],
                             sem0)
            pltpu.make_async_copy(tab.at[pl.ds(0, _CH)],
                                  buf.at[pl.ds(0, _CH)], sem0).wait()
            off = pl.multiple_of((i % 3) * _CH, _CH)
            pltpu.async_copy(buf.at[pl.ds(off, _CH)], acc.at[dst_v.at[i]],
                             sem1, add=True)
            return carry

        lax.fori_loop(0, _NCH, body, 0)
        pltpu.make_async_copy(buf.at[pl.ds(0, _CH)],
                              acc.at[pl.ds(0, _CH)], sem1).wait()
        pltpu.make_async_copy(buf.at[pl.ds(0, _CH)],
                              acc.at[pl.ds(0, _CH)], sem1).wait()
        pltpu.make_async_copy(tab.at[pl.ds(0, _CH)],
                              buf.at[pl.ds(0, _CH)], sem0).wait()
        plsc.subcore_barrier()

        for off, sz in ((0, 128), (128, 128), (256, 56)):
            rb = s * 312 + off
            pltpu.sync_copy(acc.at[pl.ds(rb, sz)], buf.at[pl.ds(0, sz)])
            pltpu.sync_copy(buf.at[pl.ds(0, sz)], out.at[pl.ds(rb, sz)])

        @pl.when(s == _NS - 1)
        def _():
            pltpu.sync_copy(acc.at[pl.ds(4992, 8)], buf.at[pl.ds(0, 8)])
            pltpu.sync_copy(buf.at[pl.ds(0, 8)], out.at[pl.ds(4992, 8)])

    @pl.when(c == 0)
    def _():
        run(p0, dst3_0, out0)

    @pl.when(c == 1)
    def _():
        run(p1, dst3_1, out1)


def _dinv_blk(dg_ref):
    return lax.rsqrt(dg_ref[:, 0:1] + 1.0)


def _tc1_body(dg, x, w, o0, o1):
    dinv = _dinv_blk(dg)
    h = jnp.dot(x[...], w[...], preferred_element_type=jnp.float32)
    p = h * dinv
    o0[...] = p[:, :_HF]
    o1[...] = p[:, _HF:]


def _tc2_body(dg, s0, s1, p0, p1, b, w, o0, o1):
    dinv = _dinv_blk(dg)
    sfull = jnp.concatenate([s0[...], s1[...]], axis=1)
    pfull = jnp.concatenate([p0[...], p1[...]], axis=1)
    xn = jnp.maximum(dinv * (sfull + pfull) + b[...], 0.0)
    h = jnp.dot(xn, w[...], preferred_element_type=jnp.float32)
    pn = h * dinv
    o0[...] = pn[:, :_HF]
    o1[...] = pn[:, _HF:]


def _tc3_body(dg, s0, s1, p0, p1, b, lw1, lb1, lw2, lb2, out, accs):
    i = pl.program_id(0)
    dinv = _dinv_blk(dg)
    sfull = jnp.concatenate([s0[...], s1[...]], axis=1)
    pfull = jnp.concatenate([p0[...], p1[...]], axis=1)
    xn = jnp.maximum(dinv * (sfull + pfull) + b[...], 0.0)

    @pl.when(i == 0)
    def _():
        accs[...] = jnp.zeros_like(accs)

    accs[...] += jnp.sum(xn, axis=0, keepdims=True)

    @pl.when(i == _NG - 1)
    def _():
        g = accs[...] * (1.0 / _N)
        g1 = jnp.dot(g, lw1[...], preferred_element_type=jnp.float32) + lb1[...]
        g1 = jnp.maximum(g1, 0.0)
        g2 = jnp.dot(g1, lw2[...], preferred_element_type=jnp.float32) + lb2[...]
        out[...] = 1.0 / (1.0 + jnp.exp(-g2))


def _row_spec(cols):
    return pl.BlockSpec((_BR, cols), lambda i: (i, 0))


def _fix_spec(shape):
    return pl.BlockSpec(shape, lambda i: (0, 0))


def _tc1(dg, x, w):
    return pl.pallas_call(
        _tc1_body,
        grid=(_NG,),
        in_specs=[_row_spec(_HF), _row_spec(_D), _fix_spec((_D, _D))],
        out_specs=[_row_spec(_HF), _row_spec(_HF)],
        out_shape=[jax.ShapeDtypeStruct((_N, _HF), jnp.float32)] * 2,
    )(dg, x, w)


def _tc2(dg, s0, s1, p0, p1, b, w):
    return pl.pallas_call(
        _tc2_body,
        grid=(_NG,),
        in_specs=[_row_spec(_HF),
                  _row_spec(_HF), _row_spec(_HF),
                  _row_spec(_HF), _row_spec(_HF),
                  _fix_spec((1, _D)), _fix_spec((_D, _D))],
        out_specs=[_row_spec(_HF), _row_spec(_HF)],
        out_shape=[jax.ShapeDtypeStruct((_N, _HF), jnp.float32)] * 2,
    )(dg, s0, s1, p0, p1, b, w)


def _tc3(dg, s0, s1, p0, p1, b, lw1, lb1, lw2, lb2):
    return pl.pallas_call(
        _tc3_body,
        grid=(_NG,),
        in_specs=[_row_spec(_HF),
                  _row_spec(_HF), _row_spec(_HF),
                  _row_spec(_HF), _row_spec(_HF),
                  _fix_spec((1, _D)), _fix_spec((_D, _D)), _fix_spec((1, _D)),
                  _fix_spec((_D, 1)), _fix_spec((1, 1))],
        out_specs=_fix_spec((1, 1)),
        out_shape=jax.ShapeDtypeStruct((1, 1), jnp.float32),
        scratch_shapes=[pltpu.VMEM((1, _D), jnp.float32)],
    )(dg, s0, s1, p0, p1, b, lw1, lb1, lw2, lb2)


def kernel(x, edge_index, W1, b1, W2, b2, W3, b3, LW1, Lb1, LW2, Lb2):
    src = edge_index[0].astype(jnp.int32)
    dst = edge_index[1].astype(jnp.int32)
    pad = _EP - _E
    srcp = jnp.concatenate([src, jnp.zeros((pad,), jnp.int32)])
    dstp = jnp.concatenate([dst, jnp.full((pad,), 2 * _NH, jnp.int32)])
    srcp, dstp = lax.sort((srcp, dstp), dimension=0, num_keys=1)
    trash = _TRASH + jnp.arange(_EP, dtype=jnp.int32) % (_ACC - _NH)
    dsta = jnp.where(dstp < _NH, dstp, trash)
    dstb0 = dstp - _NH
    dstb = jnp.where((dstb0 >= 0) & (dstb0 < _NH), dstb0, trash)
    src3 = srcp.reshape(_NS, _NCH, _CH)
    dsta3 = dsta.reshape(_NS, _NCH, _CH)
    dstb3 = dstb.reshape(_NS, _NCH, _CH)

    def scatter_full(p0, p1):
        a0, a1 = _sc_scatter(p0, p1, src3, dsta3, dsta3)
        b0, b1 = _sc_scatter(p0, p1, src3, dstb3, dstb3)
        return (jnp.concatenate([a0, b0], axis=0),
                jnp.concatenate([a1, b1], axis=0))

    ones = jnp.minimum(jnp.abs(x[:, :_HF]), 0.0) + 1.0
    seqsrc3 = (jnp.minimum(src3, 0) +
               (jnp.arange(_EP, dtype=jnp.int32) % _N).reshape(_NS, _NCH, _CH))
    dga, dgb = _sc_scatter(ones, ones, seqsrc3, dsta3, dstb3)
    dg = jnp.concatenate([dga, dgb], axis=0)

    p0, p1 = _tc1(dg, x, W1)
    s0, s1 = scatter_full(p0, p1)
    p0, p1 = _tc2(dg, s0, s1, p0, p1, b1.reshape(1, _D), W2)
    s0, s1 = scatter_full(p0, p1)
    p0, p1 = _tc2(dg, s0, s1, p0, p1, b2.reshape(1, _D), W3)
    s0, s1 = scatter_full(p0, p1)
    out = _tc3(dg, s0, s1, p0, p1, b3.reshape(1, _D),
               LW1, Lb1.reshape(1, _D), LW2, Lb2.reshape(1, 1))
    return out.reshape(1)


# speedup vs baseline: 1.8012x; 1.8012x over previous
import functools

import jax
import jax.numpy as jnp
from jax import lax
from jax.experimental import pallas as pl
from jax.experimental.pallas import tpu as pltpu
from jax.experimental.pallas import tpu_sc as plsc

_N = 10000
_D = 256
_HF = 128
_E = 160000
_CH = 128
_NS = 16
_NCH = 79
_EP = _NS * _NCH * _CH
_NH = 5000
_ACC = 5120
_TRASH = _NH

_BR = 1000
_NG = _N // _BR


def _fill(buf, val, rows, cols):
    npc = cols // 16
    v = jnp.full((16,), val, jnp.float32)

    def body(i, carry):
        r = i // npc
        co = pl.multiple_of((i % npc) * 16, 16)
        buf[r, pl.ds(co, 16)] = v
        return carry

    lax.fori_loop(0, rows * npc, body, 0)


@functools.cache
def _sc_scatter_kernel():
    return functools.partial(
        pl.kernel,
        mesh=plsc.VectorSubcoreMesh(core_axis_name="c", subcore_axis_name="s"),
        out_type=[jax.ShapeDtypeStruct((_NH, _HF), jnp.float32),
                  jax.ShapeDtypeStruct((_NH, _HF), jnp.float32)],
        scratch_types=[
            pltpu.VMEM((_NCH, _CH), jnp.int32),
            pltpu.VMEM((_NCH, _CH), jnp.int32),
            pltpu.VMEM((3 * _CH, _HF), jnp.float32),
            pltpu.SemaphoreType.DMA,
            pltpu.VMEM_SHARED((_ACC, _HF), jnp.float32),
            pltpu.SemaphoreType.DMA,
        ],
    )(_sc_scatter_body)


def _sc_scatter(p0, p1, src3, dst3_0, dst3_1):
    return _sc_scatter_kernel()(p0, p1, src3, dst3_0, dst3_1)


def _sc_scatter_body(p0, p1, src3, dst3_0, dst3_1, out0, out1,
                     src_v, dst_v, buf, sem1, acc, sem0):
    c = lax.axis_index("c")
    s = lax.axis_index("s")

    pltpu.sync_copy(src3.at[s], src_v)

    _fill(buf, 0.0, _CH, _HF)
    zb = s * 320
    for off, sz in ((0, 128), (128, 128), (256, 64)):
        pltpu.sync_copy(buf.at[pl.ds(0, sz)], acc.at[pl.ds(zb + off, sz)])
    plsc.subcore_barrier()

    def run(tab, dst3, out):
        pltpu.sync_copy(dst3.at[s], dst_v)
        pltpu.async_copy(tab.at[src_v.at[0]], buf.at[pl.ds(0, _CH)], sem0)

        def body(i, carry):
            @pl.when(i >= 2)
            def _():
                pltpu.make_async_copy(buf.at[pl.ds(0, _CH)],
                                      acc.at[pl.ds(0, _CH)], sem1).wait()
            nxt = jnp.minimum(i + 1, _NCH - 1)
            off_n = pl.multiple_of(((i + 1) % 3) * _CH, _CH)
            pltpu.async_copy(tab.at[src_v.at[nxt]], buf.at[pl.ds(off_n, _CH)],
                             sem0)
            pltpu.make_async_copy(tab.at[pl.ds(0, _CH)],
                                  buf.at[pl.ds(0, _CH)], sem0).wait()
            off = pl.multiple_of((i % 3) * _CH, _CH)
            pltpu.async_copy(buf.at[pl.ds(off, _CH)], acc.at[dst_v.at[i]],
                             sem1, add=True)
            return carry

        lax.fori_loop(0, _NCH, body, 0)
        pltpu.make_async_copy(buf.at[pl.ds(0, _CH)],
                              acc.at[pl.ds(0, _CH)], sem1).wait()
        pltpu.make_async_copy(buf.at[pl.ds(0, _CH)],
                              acc.at[pl.ds(0, _CH)], sem1).wait()
        pltpu.make_async_copy(tab.at[pl.ds(0, _CH)],
                              buf.at[pl.ds(0, _CH)], sem0).wait()
        plsc.subcore_barrier()

        for off, sz in ((0, 128), (128, 128), (256, 56)):
            rb = s * 312 + off
            pltpu.sync_copy(acc.at[pl.ds(rb, sz)], buf.at[pl.ds(0, sz)])
            pltpu.sync_copy(buf.at[pl.ds(0, sz)], out.at[pl.ds(rb, sz)])

        @pl.when(s == _NS - 1)
        def _():
            pltpu.sync_copy(acc.at[pl.ds(4992, 8)], buf.at[pl.ds(0, 8)])
            pltpu.sync_copy(buf.at[pl.ds(0, 8)], out.at[pl.ds(4992, 8)])

    @pl.when(c == 0)
    def _():
        run(p0, dst3_0, out0)

    @pl.when(c == 1)
    def _():
        run(p1, dst3_1, out1)


def _dinv_blk(dg_ref):
    return lax.rsqrt(dg_ref[:, 0:1] + 1.0)


def _tc1_body(dg, x, w, o0, o1):
    dinv = _dinv_blk(dg)
    h = jnp.dot(x[...], w[...], preferred_element_type=jnp.float32)
    p = h * dinv
    o0[...] = p[:, :_HF]
    o1[...] = p[:, _HF:]


def _tc2_body(dg, s0, s1, p0, p1, b, w, o0, o1):
    dinv = _dinv_blk(dg)
    sfull = jnp.concatenate([s0[...], s1[...]], axis=1)
    pfull = jnp.concatenate([p0[...], p1[...]], axis=1)
    xn = jnp.maximum(dinv * (sfull + pfull) + b[...], 0.0)
    h = jnp.dot(xn, w[...], preferred_element_type=jnp.float32)
    pn = h * dinv
    o0[...] = pn[:, :_HF]
    o1[...] = pn[:, _HF:]


def _tc3_body(dg, s0, s1, p0, p1, b, lw1, lb1, lw2, lb2, out, accs):
    i = pl.program_id(0)
    dinv = _dinv_blk(dg)
    sfull = jnp.concatenate([s0[...], s1[...]], axis=1)
    pfull = jnp.concatenate([p0[...], p1[...]], axis=1)
    xn = jnp.maximum(dinv * (sfull + pfull) + b[...], 0.0)

    @pl.when(i == 0)
    def _():
        accs[...] = jnp.zeros_like(accs)

    accs[...] += jnp.sum(xn, axis=0, keepdims=True)

    @pl.when(i == _NG - 1)
    def _():
        g = accs[...] * (1.0 / _N)
        g1 = jnp.dot(g, lw1[...], preferred_element_type=jnp.float32) + lb1[...]
        g1 = jnp.maximum(g1, 0.0)
        g2 = jnp.dot(g1, lw2[...], preferred_element_type=jnp.float32) + lb2[...]
        out[...] = 1.0 / (1.0 + jnp.exp(-g2))


def _row_spec(cols):
    return pl.BlockSpec((_BR, cols), lambda i: (i, 0))


def _fix_spec(shape):
    return pl.BlockSpec(shape, lambda i: (0, 0))


def _tc1(dg, x, w):
    return pl.pallas_call(
        _tc1_body,
        grid=(_NG,),
        in_specs=[_row_spec(_HF), _row_spec(_D), _fix_spec((_D, _D))],
        out_specs=[_row_spec(_HF), _row_spec(_HF)],
        out_shape=[jax.ShapeDtypeStruct((_N, _HF), jnp.float32)] * 2,
    )(dg, x, w)


def _tc2(dg, s0, s1, p0, p1, b, w):
    return pl.pallas_call(
        _tc2_body,
        grid=(_NG,),
        in_specs=[_row_spec(_HF),
                  _row_spec(_HF), _row_spec(_HF),
                  _row_spec(_HF), _row_spec(_HF),
                  _fix_spec((1, _D)), _fix_spec((_D, _D))],
        out_specs=[_row_spec(_HF), _row_spec(_HF)],
        out_shape=[jax.ShapeDtypeStruct((_N, _HF), jnp.float32)] * 2,
    )(dg, s0, s1, p0, p1, b, w)


def _tc3(dg, s0, s1, p0, p1, b, lw1, lb1, lw2, lb2):
    return pl.pallas_call(
        _tc3_body,
        grid=(_NG,),
        in_specs=[_row_spec(_HF),
                  _row_spec(_HF), _row_spec(_HF),
                  _row_spec(_HF), _row_spec(_HF),
                  _fix_spec((1, _D)), _fix_spec((_D, _D)), _fix_spec((1, _D)),
                  _fix_spec((_D, 1)), _fix_spec((1, 1))],
        out_specs=_fix_spec((1, 1)),
        out_shape=jax.ShapeDtypeStruct((1, 1), jnp.float32),
        scratch_shapes=[pltpu.VMEM((1, _D), jnp.float32)],
    )(dg, s0, s1, p0, p1, b, lw1, lb1, lw2, lb2)


def kernel(x, edge_index, W1, b1, W2, b2, W3, b3, LW1, Lb1, LW2, Lb2):
    src = edge_index[0].astype(jnp.int32)
    dst = edge_index[1].astype(jnp.int32)
    pad = _EP - _E
    srcp = jnp.concatenate([src, jnp.zeros((pad,), jnp.int32)])
    dstp = jnp.concatenate([dst, jnp.full((pad,), 2 * _NH, jnp.int32)])
    trash = _TRASH + jnp.arange(_EP, dtype=jnp.int32) % (_ACC - _NH)
    dsta = jnp.where(dstp < _NH, dstp, trash)
    dstb0 = dstp - _NH
    dstb = jnp.where((dstb0 >= 0) & (dstb0 < _NH), dstb0, trash)
    src3 = srcp.reshape(_NS, _NCH, _CH)
    dsta3 = dsta.reshape(_NS, _NCH, _CH)
    dstb3 = dstb.reshape(_NS, _NCH, _CH)

    def scatter_full(p0, p1):
        a0, a1 = _sc_scatter(p0, p1, src3, dsta3, dsta3)
        b0, b1 = _sc_scatter(p0, p1, src3, dstb3, dstb3)
        return (jnp.concatenate([a0, b0], axis=0),
                jnp.concatenate([a1, b1], axis=0))

    ones = jnp.minimum(jnp.abs(x[:, :_HF]), 0.0) + 1.0
    seqsrc3 = (jnp.minimum(src3, 0) +
               (jnp.arange(_EP, dtype=jnp.int32) % _N).reshape(_NS, _NCH, _CH))
    dga, dgb = _sc_scatter(ones, ones, seqsrc3, dsta3, dstb3)
    dg = jnp.concatenate([dga, dgb], axis=0)

    p0, p1 = _tc1(dg, x, W1)
    s0, s1 = scatter_full(p0, p1)
    p0, p1 = _tc2(dg, s0, s1, p0, p1, b1.reshape(1, _D), W2)
    s0, s1 = scatter_full(p0, p1)
    p0, p1 = _tc2(dg, s0, s1, p0, p1, b2.reshape(1, _D), W3)
    s0, s1 = scatter_full(p0, p1)
    out = _tc3(dg, s0, s1, p0, p1, b3.reshape(1, _D),
               LW1, Lb1.reshape(1, _D), LW2, Lb2.reshape(1, 1))
    return out.reshape(1)
